# baseline (speedup 1.0000x reference)
import functools

import jax
import jax.numpy as jnp
from jax import lax
from jax.experimental import pallas as pl
from jax.experimental.pallas import tpu as pltpu
from jax.experimental.pallas import tpu_sc as plsc

N = 10000
E = 320000
D = 128

NP = 10240
NC = 2
NS = 16
NW = NC * NS
CHUNK = 128
NCHUNK = 2 * (((E + NW * CHUNK - 1) // (NW * CHUNK) + 1) // 2)
QUOTA = NCHUNK * CHUNK
EPAD = QUOTA * NW
IDXLEN = EPAD + CHUNK
STRIPE = NP // NS


def _tables_body(emb_ref, d_ref, w1_ref, w2_ref, b_ref, a_ref, s_ref, e2_ref):
    b = b_ref[0]
    emb = emb_ref[...]
    a_ref[...] = jnp.sum(emb * w1_ref[...], axis=1) + b
    s_ref[...] = jnp.sum(emb * w2_ref[...], axis=1)
    e2_ref[...] = emb * d_ref[...][:, None]


def _make_tables(emb_pad, d_pad, w1, w2, b):
    blk = 2048
    grid = (NP // blk,)
    return pl.pallas_call(
        _tables_body,
        grid=grid,
        in_specs=[
            pl.BlockSpec((blk, D), lambda i: (i, 0)),
            pl.BlockSpec((blk,), lambda i: (i,)),
            pl.BlockSpec((1, D), lambda i: (0, 0)),
            pl.BlockSpec((1, D), lambda i: (0, 0)),
            pl.BlockSpec(memory_space=pltpu.SMEM),
        ],
        out_specs=[
            pl.BlockSpec((blk,), lambda i: (i,)),
            pl.BlockSpec((blk,), lambda i: (i,)),
            pl.BlockSpec((blk, D), lambda i: (i, 0)),
        ],
        out_shape=[
            jax.ShapeDtypeStruct((NP,), jnp.float32),
            jax.ShapeDtypeStruct((NP,), jnp.float32),
            jax.ShapeDtypeStruct((NP, D), jnp.float32),
        ],
    )(emb_pad, d_pad, w1, w2, b)


def _sum_body(p0_ref, p1_ref, d_ref, o_ref):
    o_ref[...] = (p0_ref[...] + p1_ref[...]) * d_ref[...][:, None]


def _sum_partials(zparts, d_pad):
    blk = 2048
    return pl.pallas_call(
        _sum_body,
        grid=(NP // blk,),
        in_specs=[
            pl.BlockSpec((blk, D), lambda i: (i, 0)),
            pl.BlockSpec((blk, D), lambda i: (i, 0)),
            pl.BlockSpec((blk,), lambda i: (i,)),
        ],
        out_specs=pl.BlockSpec((blk, D), lambda i: (i, 0)),
        out_shape=jax.ShapeDtypeStruct((NP, D), jnp.float32),
    )(zparts[0], zparts[1], d_pad)


def _sc_body(emb2_hbm, a_hbm, s_hbm, src_hbm, dst_hbm, zout_hbm,
             src0, src1, dst0, dst1, dsts0, dsts1, rows0, rows1,
             av_v, sv_v, evals_v, z_sh, semi0, semi1, semg, sema, semb,
             sems0, sems1):
    cid = lax.axis_index("c")
    sid = lax.axis_index("s")
    wid = cid * NS + sid
    srcs = (src0, src1)
    dsts_in = (dst0, dst1)
    dsts_sc = (dsts0, dsts1)
    rows = (rows0, rows1)
    semi = (semi0, semi1)
    sems = (sems0, sems1)

    zero = jnp.zeros((16,), jnp.float32)

    def zero_row(r, carry):
        for k in range(D // 16):
            rows0[r, pl.ds(k * 16, 16)] = zero
        return carry

    lax.fori_loop(0, CHUNK, zero_row, 0)
    for i in range(STRIPE // CHUNK):
        pltpu.sync_copy(rows0, z_sh.at[pl.ds(sid * STRIPE + i * CHUNK, CHUNK)])
    plsc.subcore_barrier()

    base0 = wid * QUOTA

    def issue_idx(c, b):
        base = base0 + c * CHUNK
        pltpu.async_copy(src_hbm.at[pl.ds(base, CHUNK)], srcs[b], semi[b])
        pltpu.async_copy(dst_hbm.at[pl.ds(base, CHUNK)], dsts_in[b], semi[b])

    def wait_idx(c, b):
        base = base0 + c * CHUNK
        pltpu.make_async_copy(src_hbm.at[pl.ds(base, CHUNK)], srcs[b],
                              semi[b]).wait()
        pltpu.make_async_copy(dst_hbm.at[pl.ds(base, CHUNK)], dsts_in[b],
                              semi[b]).wait()

    def do_chunk(c, b):
        @pl.when(c >= 2)
        def _():
            pltpu.make_async_copy(rows[b], z_sh.at[dsts_sc[b]], sems[b]).wait()

        wait_idx(c, b)
        pltpu.async_copy(emb2_hbm.at[srcs[b]], rows[b], semg)
        pltpu.async_copy(a_hbm.at[dsts_in[b]], av_v, sema)
        pltpu.async_copy(s_hbm.at[srcs[b]], sv_v, semb)
        issue_idx(c + 1, 1 - b)
        for g in range(CHUNK // 16):
            dsts_sc[b][pl.ds(g * 16, 16)] = dsts_in[b][pl.ds(g * 16, 16)]
        pltpu.make_async_copy(a_hbm.at[dsts_in[b]], av_v, sema).wait()
        pltpu.make_async_copy(s_hbm.at[srcs[b]], sv_v, semb).wait()
        for g in range(CHUNK // 16):
            t = av_v[pl.ds(g * 16, 16)] + sv_v[pl.ds(g * 16, 16)]
            evals_v[pl.ds(g * 16, 16)] = 1.0 - 2.0 / (jnp.exp(2.0 * t) + 1.0)
        pltpu.make_async_copy(emb2_hbm.at[srcs[b]], rows[b], semg).wait()

        @pl.loop(0, CHUNK // 16)
        def scale_group(g):
            e_grp = evals_v[pl.ds(g * 16, 16)]
            for j in range(16):
                e_spl = e_grp.at[jnp.full((16,), j, jnp.int32)].get(
                    mode="promise_in_bounds")
                r = g * 16 + j
                for k in range(D // 16):
                    rows[b][r, pl.ds(k * 16, 16)] = (
                        rows[b][r, pl.ds(k * 16, 16)] * e_spl)

        pltpu.async_copy(rows[b], z_sh.at[dsts_sc[b]], sems[b], add=True)

    issue_idx(0, 0)

    @pl.loop(0, NCHUNK, step=2)
    def pipeline(c):
        do_chunk(c, 0)
        do_chunk(c + 1, 1)

    wait_idx(NCHUNK, 0)
    pltpu.make_async_copy(rows[0], z_sh.at[dsts_sc[0]], sems[0]).wait()
    pltpu.make_async_copy(rows[1], z_sh.at[dsts_sc[1]], sems[1]).wait()

    plsc.subcore_barrier()
    pltpu.sync_copy(
        z_sh.at[pl.ds(sid * STRIPE, STRIPE)],
        zout_hbm.at[pl.ds(cid * NP + sid * STRIPE, STRIPE)],
    )


@functools.partial(
    pl.kernel,
    mesh=plsc.VectorSubcoreMesh(core_axis_name="c", subcore_axis_name="s"),
    out_type=jax.ShapeDtypeStruct((NC * NP, D), jnp.float32),
    compiler_params=pltpu.CompilerParams(needs_layout_passes=False),
    scratch_types=[
        pltpu.VMEM((CHUNK,), jnp.int32),
        pltpu.VMEM((CHUNK,), jnp.int32),
        pltpu.VMEM((CHUNK,), jnp.int32),
        pltpu.VMEM((CHUNK,), jnp.int32),
        pltpu.VMEM((CHUNK,), jnp.int32),
        pltpu.VMEM((CHUNK,), jnp.int32),
        pltpu.VMEM((CHUNK, D), jnp.float32),
        pltpu.VMEM((CHUNK, D), jnp.float32),
        pltpu.VMEM((CHUNK,), jnp.float32),
        pltpu.VMEM((CHUNK,), jnp.float32),
        pltpu.VMEM((CHUNK,), jnp.float32),
        pltpu.VMEM_SHARED((NP, D), jnp.float32),
        pltpu.SemaphoreType.DMA,
        pltpu.SemaphoreType.DMA,
        pltpu.SemaphoreType.DMA,
        pltpu.SemaphoreType.DMA,
        pltpu.SemaphoreType.DMA,
        pltpu.SemaphoreType.DMA,
        pltpu.SemaphoreType.DMA,
    ],
)
def _sc_aggregate(*args):
    _sc_body(*args)


@jax.jit
def kernel(emb, d, edge_index, gate_W, gate_b):
    emb_pad = jnp.zeros((NP, D), jnp.float32).at[:N].set(emb)
    d_pad = jnp.zeros((NP,), jnp.float32).at[:N].set(d)
    w1 = gate_W[:, :D]
    w2 = gate_W[:, D:]
    a_tbl, s_tbl, emb2 = _make_tables(emb_pad, d_pad, w1, w2, gate_b)
    pad = jnp.full((IDXLEN - E,), N, jnp.int32)
    src = jnp.concatenate([edge_index[0], pad])
    dst = jnp.concatenate([edge_index[1], pad])
    zparts = _sc_aggregate(emb2, a_tbl, s_tbl, src, dst)
    z = _sum_partials(zparts.reshape(NC, NP, D), d_pad)
    return z[:N]


# speedup vs baseline: 1.4290x; 1.4290x over previous
import functools

import jax
import jax.numpy as jnp
from jax import lax
from jax.experimental import pallas as pl
from jax.experimental.pallas import tpu as pltpu
from jax.experimental.pallas import tpu_sc as plsc

N = 10000
E = 320000
D = 128

NP = 10240
NC = 2
NS = 16
NW = NC * NS
CHUNK = 96
NCHUNK = 2 * (((E + NW * CHUNK - 1) // (NW * CHUNK) + 1) // 2)
QUOTA = NCHUNK * CHUNK
EPAD = QUOTA * NW
IDXLEN = EPAD + CHUNK
STRIPE = NP // NS


def _tables_body(emb_ref, d_ref, w1_ref, w2_ref, b_ref, a_ref, s_ref, e2_ref):
    b = b_ref[0]
    emb = emb_ref[...]
    a_ref[...] = jnp.sum(emb * w1_ref[...], axis=1) + b
    s_ref[...] = jnp.sum(emb * w2_ref[...], axis=1)
    e2_ref[...] = emb * d_ref[...][:, None]


def _make_tables(emb_pad, d_pad, w1, w2, b):
    blk = 2048
    grid = (NP // blk,)
    return pl.pallas_call(
        _tables_body,
        grid=grid,
        in_specs=[
            pl.BlockSpec((blk, D), lambda i: (i, 0)),
            pl.BlockSpec((blk,), lambda i: (i,)),
            pl.BlockSpec((1, D), lambda i: (0, 0)),
            pl.BlockSpec((1, D), lambda i: (0, 0)),
            pl.BlockSpec(memory_space=pltpu.SMEM),
        ],
        out_specs=[
            pl.BlockSpec((blk,), lambda i: (i,)),
            pl.BlockSpec((blk,), lambda i: (i,)),
            pl.BlockSpec((blk, D), lambda i: (i, 0)),
        ],
        out_shape=[
            jax.ShapeDtypeStruct((NP,), jnp.float32),
            jax.ShapeDtypeStruct((NP,), jnp.float32),
            jax.ShapeDtypeStruct((NP, D), jnp.float32),
        ],
    )(emb_pad, d_pad, w1, w2, b)


def _sum_body(p0_ref, p1_ref, d_ref, o_ref):
    o_ref[...] = (p0_ref[...] + p1_ref[...]) * d_ref[...][:, None]


def _sum_partials(zparts, d_pad):
    blk = 2048
    return pl.pallas_call(
        _sum_body,
        grid=(NP // blk,),
        in_specs=[
            pl.BlockSpec((blk, D), lambda i: (i, 0)),
            pl.BlockSpec((blk, D), lambda i: (i, 0)),
            pl.BlockSpec((blk,), lambda i: (i,)),
        ],
        out_specs=pl.BlockSpec((blk, D), lambda i: (i, 0)),
        out_shape=jax.ShapeDtypeStruct((NP, D), jnp.float32),
    )(zparts[0], zparts[1], d_pad)


def _sc_body(emb2_hbm, a_hbm, s_hbm, src_hbm, dst_hbm, zout_hbm,
             a_t, s_t, src0, src1, dst0, dst1, dsts0, dsts1, rows0, rows1,
             evals_v, z_sh, semi0, semi1, semg, sems0, sems1):
    cid = lax.axis_index("c")
    sid = lax.axis_index("s")
    wid = cid * NS + sid
    srcs = (src0, src1)
    dsts_in = (dst0, dst1)
    dsts_sc = (dsts0, dsts1)
    rows = (rows0, rows1)
    semi = (semi0, semi1)
    sems = (sems0, sems1)

    pltpu.sync_copy(a_hbm, a_t)
    pltpu.sync_copy(s_hbm, s_t)

    zero = jnp.zeros((16,), jnp.float32)

    def zero_row(r, carry):
        for k in range(D // 16):
            rows0[r, pl.ds(k * 16, 16)] = zero
        return carry

    lax.fori_loop(0, CHUNK, zero_row, 0)
    nfull = STRIPE // CHUNK
    for i in range(nfull):
        pltpu.sync_copy(rows0, z_sh.at[pl.ds(sid * STRIPE + i * CHUNK, CHUNK)])
    rem = STRIPE - nfull * CHUNK
    if rem:
        pltpu.sync_copy(rows0.at[pl.ds(0, rem)],
                        z_sh.at[pl.ds(sid * STRIPE + nfull * CHUNK, rem)])
    plsc.subcore_barrier()

    base0 = wid * QUOTA

    def issue_idx(c, b):
        base = base0 + c * CHUNK
        pltpu.async_copy(src_hbm.at[pl.ds(base, CHUNK)], srcs[b], semi[b])
        pltpu.async_copy(dst_hbm.at[pl.ds(base, CHUNK)], dsts_in[b], semi[b])

    def wait_idx(c, b):
        base = base0 + c * CHUNK
        pltpu.make_async_copy(src_hbm.at[pl.ds(base, CHUNK)], srcs[b],
                              semi[b]).wait()
        pltpu.make_async_copy(dst_hbm.at[pl.ds(base, CHUNK)], dsts_in[b],
                              semi[b]).wait()

    def do_chunk(c, b):
        @pl.when(c >= 2)
        def _():
            pltpu.make_async_copy(rows[b], z_sh.at[dsts_sc[b]], sems[b]).wait()

        wait_idx(c, b)
        pltpu.async_copy(emb2_hbm.at[srcs[b]], rows[b], semg)
        issue_idx(c + 1, 1 - b)
        for g in range(CHUNK // 16):
            sv = srcs[b][pl.ds(g * 16, 16)]
            dv = dsts_in[b][pl.ds(g * 16, 16)]
            t = plsc.load_gather(a_t, [dv]) + plsc.load_gather(s_t, [sv])
            evals_v[pl.ds(g * 16, 16)] = 1.0 - 2.0 / (jnp.exp(2.0 * t) + 1.0)
            dsts_sc[b][pl.ds(g * 16, 16)] = dv
        pltpu.make_async_copy(emb2_hbm.at[srcs[b]], rows[b], semg).wait()

        @pl.loop(0, CHUNK // 16)
        def scale_group(g):
            e_grp = evals_v[pl.ds(g * 16, 16)]
            for j in range(16):
                e_spl = e_grp.at[jnp.full((16,), j, jnp.int32)].get(
                    mode="promise_in_bounds")
                r = g * 16 + j
                for k in range(D // 16):
                    rows[b][r, pl.ds(k * 16, 16)] = (
                        rows[b][r, pl.ds(k * 16, 16)] * e_spl)

        pltpu.async_copy(rows[b], z_sh.at[dsts_sc[b]], sems[b], add=True)

    issue_idx(0, 0)

    @pl.loop(0, NCHUNK, step=2)
    def pipeline(c):
        do_chunk(c, 0)
        do_chunk(c + 1, 1)

    wait_idx(NCHUNK, 0)
    pltpu.make_async_copy(rows[0], z_sh.at[dsts_sc[0]], sems[0]).wait()
    pltpu.make_async_copy(rows[1], z_sh.at[dsts_sc[1]], sems[1]).wait()

    plsc.subcore_barrier()
    pltpu.sync_copy(
        z_sh.at[pl.ds(sid * STRIPE, STRIPE)],
        zout_hbm.at[pl.ds(cid * NP + sid * STRIPE, STRIPE)],
    )


@functools.partial(
    pl.kernel,
    mesh=plsc.VectorSubcoreMesh(core_axis_name="c", subcore_axis_name="s"),
    out_type=jax.ShapeDtypeStruct((NC * NP, D), jnp.float32),
    compiler_params=pltpu.CompilerParams(needs_layout_passes=False),
    scratch_types=[
        pltpu.VMEM((NP,), jnp.float32),
        pltpu.VMEM((NP,), jnp.float32),
        pltpu.VMEM((CHUNK,), jnp.int32),
        pltpu.VMEM((CHUNK,), jnp.int32),
        pltpu.VMEM((CHUNK,), jnp.int32),
        pltpu.VMEM((CHUNK,), jnp.int32),
        pltpu.VMEM((CHUNK,), jnp.int32),
        pltpu.VMEM((CHUNK,), jnp.int32),
        pltpu.VMEM((CHUNK, D), jnp.float32),
        pltpu.VMEM((CHUNK, D), jnp.float32),
        pltpu.VMEM((CHUNK,), jnp.float32),
        pltpu.VMEM_SHARED((NP, D), jnp.float32),
        pltpu.SemaphoreType.DMA,
        pltpu.SemaphoreType.DMA,
        pltpu.SemaphoreType.DMA,
        pltpu.SemaphoreType.DMA,
        pltpu.SemaphoreType.DMA,
    ],
)
def _sc_aggregate(*args):
    _sc_body(*args)


@jax.jit
def kernel(emb, d, edge_index, gate_W, gate_b):
    emb_pad = jnp.zeros((NP, D), jnp.float32).at[:N].set(emb)
    d_pad = jnp.zeros((NP,), jnp.float32).at[:N].set(d)
    w1 = gate_W[:, :D]
    w2 = gate_W[:, D:]
    a_tbl, s_tbl, emb2 = _make_tables(emb_pad, d_pad, w1, w2, gate_b)
    pad = jnp.full((IDXLEN - E,), N, jnp.int32)
    src = jnp.concatenate([edge_index[0], pad])
    dst = jnp.concatenate([edge_index[1], pad])
    zparts = _sc_aggregate(emb2, a_tbl, s_tbl, src, dst)
    z = _sum_partials(zparts.reshape(NC, NP, D), d_pad)
    return z[:N]


# speedup vs baseline: 2.0014x; 1.4006x over previous
import functools

import jax
import jax.numpy as jnp
from jax import lax
from jax.experimental import pallas as pl
from jax.experimental.pallas import tpu as pltpu
from jax.experimental.pallas import tpu_sc as plsc

N = 10000
E = 320000
D = 128

NP = 10240
NC = 2
NS = 16
NW = NC * NS
CHUNK = 96
Q0 = 140
Q1 = 70
EPAD = NS * (Q0 + Q1) * CHUNK
IDXLEN = EPAD + CHUNK
STRIPE = NP // NS


def _tables_body(emb_ref, d_ref, w1_ref, w2_ref, b_ref, a_ref, s_ref, e2_ref):
    b = b_ref[0]
    emb = emb_ref[...]
    a_ref[...] = jnp.sum(emb * w1_ref[...], axis=1) + b
    s_ref[...] = jnp.sum(emb * w2_ref[...], axis=1)
    e2_ref[...] = emb * d_ref[...][:, None]


def _make_tables(emb_pad, d_pad, w1, w2, b):
    blk = 2048
    grid = (NP // blk,)
    return pl.pallas_call(
        _tables_body,
        grid=grid,
        in_specs=[
            pl.BlockSpec((blk, D), lambda i: (i, 0)),
            pl.BlockSpec((blk,), lambda i: (i,)),
            pl.BlockSpec((1, D), lambda i: (0, 0)),
            pl.BlockSpec((1, D), lambda i: (0, 0)),
            pl.BlockSpec(memory_space=pltpu.SMEM),
        ],
        out_specs=[
            pl.BlockSpec((blk,), lambda i: (i,)),
            pl.BlockSpec((blk,), lambda i: (i,)),
            pl.BlockSpec((blk, D), lambda i: (i, 0)),
        ],
        out_shape=[
            jax.ShapeDtypeStruct((NP,), jnp.float32),
            jax.ShapeDtypeStruct((NP,), jnp.float32),
            jax.ShapeDtypeStruct((NP, D), jnp.float32),
        ],
    )(emb_pad, d_pad, w1, w2, b)


def _sum_body(p0_ref, p1_ref, d_ref, o_ref):
    o_ref[...] = (p0_ref[...] + p1_ref[...]) * d_ref[...][:, None]


def _sum_partials(zparts, d_pad):
    blk = 2048
    return pl.pallas_call(
        _sum_body,
        grid=(NP // blk,),
        in_specs=[
            pl.BlockSpec((blk, D), lambda i: (i, 0)),
            pl.BlockSpec((blk, D), lambda i: (i, 0)),
            pl.BlockSpec((blk,), lambda i: (i,)),
        ],
        out_specs=pl.BlockSpec((blk, D), lambda i: (i, 0)),
        out_shape=jax.ShapeDtypeStruct((NP, D), jnp.float32),
    )(zparts[0], zparts[1], d_pad)


def _sc_body(emb2_hbm, a_hbm, s_hbm, src_hbm, dst_hbm, zout_hbm,
             a_t, s_t, src0, src1, dst0, dst1, dsts0, dsts1, rows0, rows1,
             evals_v, z_sh, semi0, semi1, semg, sems0, sems1):
    cid = lax.axis_index("c")
    sid = lax.axis_index("s")
    wid = cid * NS + sid
    srcs = (src0, src1)
    dsts_in = (dst0, dst1)
    dsts_sc = (dsts0, dsts1)
    rows = (rows0, rows1)
    semi = (semi0, semi1)
    sems = (sems0, sems1)

    pltpu.sync_copy(a_hbm, a_t)
    pltpu.sync_copy(s_hbm, s_t)

    zero = jnp.zeros((16,), jnp.float32)

    def zero_row(r, carry):
        for k in range(D // 16):
            rows0[r, pl.ds(k * 16, 16)] = zero
        return carry

    lax.fori_loop(0, CHUNK, zero_row, 0)
    nfull = STRIPE // CHUNK
    for i in range(nfull):
        pltpu.sync_copy(rows0, z_sh.at[pl.ds(sid * STRIPE + i * CHUNK, CHUNK)])
    rem = STRIPE - nfull * CHUNK
    if rem:
        pltpu.sync_copy(rows0.at[pl.ds(0, rem)],
                        z_sh.at[pl.ds(sid * STRIPE + nfull * CHUNK, rem)])
    plsc.subcore_barrier()

    nchunk = jnp.where(cid == 0, Q0, Q1)
    base0 = jnp.where(cid == 0, sid * (Q0 * CHUNK),
                      NS * (Q0 * CHUNK) + sid * (Q1 * CHUNK))

    def issue_idx(c, b):
        base = base0 + c * CHUNK
        pltpu.async_copy(src_hbm.at[pl.ds(base, CHUNK)], srcs[b], semi[b])
        pltpu.async_copy(dst_hbm.at[pl.ds(base, CHUNK)], dsts_in[b], semi[b])

    def wait_idx(c, b):
        base = base0 + c * CHUNK
        pltpu.make_async_copy(src_hbm.at[pl.ds(base, CHUNK)], srcs[b],
                              semi[b]).wait()
        pltpu.make_async_copy(dst_hbm.at[pl.ds(base, CHUNK)], dsts_in[b],
                              semi[b]).wait()

    def do_chunk(c, b):
        @pl.when(c >= 2)
        def _():
            pltpu.make_async_copy(rows[b], z_sh.at[dsts_sc[b]], sems[b]).wait()

        wait_idx(c, b)
        pltpu.async_copy(emb2_hbm.at[srcs[b]], rows[b], semg)
        issue_idx(c + 1, 1 - b)
        for g in range(CHUNK // 16):
            sv = srcs[b][pl.ds(g * 16, 16)]
            dv = dsts_in[b][pl.ds(g * 16, 16)]
            t = plsc.load_gather(a_t, [dv]) + plsc.load_gather(s_t, [sv])
            evals_v[pl.ds(g * 16, 16)] = 1.0 - 2.0 / (jnp.exp(2.0 * t) + 1.0)
            dsts_sc[b][pl.ds(g * 16, 16)] = dv
        pltpu.make_async_copy(emb2_hbm.at[srcs[b]], rows[b], semg).wait()

        @pl.loop(0, CHUNK // 16)
        def scale_group(g):
            e_grp = evals_v[pl.ds(g * 16, 16)]
            for j in range(16):
                e_spl = e_grp.at[jnp.full((16,), j, jnp.int32)].get(
                    mode="promise_in_bounds")
                r = g * 16 + j
                for k in range(D // 16):
                    rows[b][r, pl.ds(k * 16, 16)] = (
                        rows[b][r, pl.ds(k * 16, 16)] * e_spl)

        pltpu.async_copy(rows[b], z_sh.at[dsts_sc[b]], sems[b], add=True)

    issue_idx(0, 0)

    @pl.loop(0, nchunk, step=2)
    def pipeline(c):
        do_chunk(c, 0)
        do_chunk(c + 1, 1)

    wait_idx(nchunk, 0)
    pltpu.make_async_copy(rows[0], z_sh.at[dsts_sc[0]], sems[0]).wait()
    pltpu.make_async_copy(rows[1], z_sh.at[dsts_sc[1]], sems[1]).wait()

    plsc.subcore_barrier()
    pltpu.sync_copy(
        z_sh.at[pl.ds(sid * STRIPE, STRIPE)],
        zout_hbm.at[pl.ds(cid * NP + sid * STRIPE, STRIPE)],
    )


@functools.partial(
    pl.kernel,
    mesh=plsc.VectorSubcoreMesh(core_axis_name="c", subcore_axis_name="s"),
    out_type=jax.ShapeDtypeStruct((NC * NP, D), jnp.float32),
    compiler_params=pltpu.CompilerParams(needs_layout_passes=False),
    scratch_types=[
        pltpu.VMEM((NP,), jnp.float32),
        pltpu.VMEM((NP,), jnp.float32),
        pltpu.VMEM((CHUNK,), jnp.int32),
        pltpu.VMEM((CHUNK,), jnp.int32),
        pltpu.VMEM((CHUNK,), jnp.int32),
        pltpu.VMEM((CHUNK,), jnp.int32),
        pltpu.VMEM((CHUNK,), jnp.int32),
        pltpu.VMEM((CHUNK,), jnp.int32),
        pltpu.VMEM((CHUNK, D), jnp.float32),
        pltpu.VMEM((CHUNK, D), jnp.float32),
        pltpu.VMEM((CHUNK,), jnp.float32),
        pltpu.VMEM_SHARED((NP, D), jnp.float32),
        pltpu.SemaphoreType.DMA,
        pltpu.SemaphoreType.DMA,
        pltpu.SemaphoreType.DMA,
        pltpu.SemaphoreType.DMA,
        pltpu.SemaphoreType.DMA,
    ],
)
def _sc_aggregate(*args):
    _sc_body(*args)


@jax.jit
def kernel(emb, d, edge_index, gate_W, gate_b):
    emb_pad = jnp.zeros((NP, D), jnp.float32).at[:N].set(emb)
    d_pad = jnp.zeros((NP,), jnp.float32).at[:N].set(d)
    w1 = gate_W[:, :D]
    w2 = gate_W[:, D:]
    a_tbl, s_tbl, emb2 = _make_tables(emb_pad, d_pad, w1, w2, gate_b)
    pad = jnp.full((IDXLEN - E,), N, jnp.int32)
    src = jnp.concatenate([edge_index[0], pad])
    dst = jnp.concatenate([edge_index[1], pad])
    zparts = _sc_aggregate(emb2, a_tbl, s_tbl, src, dst)
    z = _sum_partials(zparts.reshape(NC, NP, D), d_pad)
    return z[:N]
